# baseline (speedup 1.0000x reference)
import jax
import jax.numpy as jnp
from jax.experimental import pallas as pl
from jax.experimental.pallas import tpu as pltpu

_BT = 128
_NBUF = 16


def _router_block(x_ref, w_ref, b_ref, o_ref, buf, sem):
    i = pl.program_id(0)
    nstep = pl.num_programs(0)

    def fetch(block, slot):
        pltpu.make_async_copy(
            x_ref.at[pl.ds(block * _BT, _BT), :], buf.at[slot], sem.at[slot],
        ).start()

    @pl.when(i == 0)
    def _prologue():
        for j in range(_NBUF - 1):
            fetch(j, j)

    nxt = i + _NBUF - 1
    slot_nxt = jax.lax.rem(nxt, _NBUF)

    @pl.when(nxt < nstep)
    def _prefetch():
        fetch(nxt, slot_nxt)

    slot = jax.lax.rem(i, _NBUF)
    pltpu.make_async_copy(
        x_ref.at[pl.ds(i * _BT, _BT), :], buf.at[slot], sem.at[slot],
    ).wait()

    xb = buf[slot].astype(jnp.bfloat16)
    wb = w_ref[...].astype(jnp.bfloat16)
    acc = jax.lax.dot_general(
        wb, xb, (((1,), (1,)), ((), ())),
        preferred_element_type=jnp.float32)
    o_ref[...] = acc + b_ref[...]


def kernel(x, W, b):
    tokens, d = x.shape
    e = W.shape[0]
    b2 = b.reshape(e, 1)
    logits_t = pl.pallas_call(
        _router_block,
        grid=(tokens // _BT,),
        in_specs=[
            pl.BlockSpec(memory_space=pltpu.MemorySpace.HBM),
            pl.BlockSpec((e, d), lambda i: (0, 0)),
            pl.BlockSpec((e, 1), lambda i: (0, 0)),
        ],
        out_specs=pl.BlockSpec((e, _BT), lambda i: (0, i)),
        out_shape=jax.ShapeDtypeStruct((e, tokens), jnp.float32),
        scratch_shapes=[
            pltpu.VMEM((_NBUF, _BT, d), jnp.float32),
            pltpu.SemaphoreType.DMA((_NBUF,)),
        ],
    )(x, W, b2)
    return logits_t.T


# speedup vs baseline: 1.0066x; 1.0066x over previous
import jax
import jax.numpy as jnp
from jax.experimental import pallas as pl

_BT = 512


def _router_block(x_ref, w_ref, b_ref, o_ref):
    xb = x_ref[...].astype(jnp.bfloat16)
    wb = w_ref[...].astype(jnp.bfloat16)
    acc = jax.lax.dot_general(
        wb, xb, (((1,), (1,)), ((), ())),
        preferred_element_type=jnp.float32)
    o_ref[...] = acc + jnp.transpose(b_ref[...], (1, 0))


def kernel(x, W, b):
    tokens, d = x.shape
    e = W.shape[0]
    b2 = b.reshape(1, e)
    logits_t = pl.pallas_call(
        _router_block,
        grid=(tokens // _BT,),
        in_specs=[
            pl.BlockSpec((_BT, d), lambda i: (i, 0)),
            pl.BlockSpec((e, d), lambda i: (0, 0)),
            pl.BlockSpec((1, e), lambda i: (0, 0)),
        ],
        out_specs=pl.BlockSpec((e, _BT), lambda i: (0, i)),
        out_shape=jax.ShapeDtypeStruct((e, tokens), jnp.float32),
    )(x, W, b2)
    return logits_t.T


# speedup vs baseline: 1.0306x; 1.0239x over previous
import jax
import jax.numpy as jnp
from jax.experimental import pallas as pl

_BT = 1024


def _router_block(x_ref, w_ref, b_ref, o_ref):
    xb = x_ref[...].astype(jnp.bfloat16)
    wb = w_ref[...].astype(jnp.bfloat16)
    acc = jax.lax.dot_general(
        wb, xb, (((1,), (1,)), ((), ())),
        preferred_element_type=jnp.float32)
    o_ref[...] = acc + jnp.transpose(b_ref[...], (1, 0))


def kernel(x, W, b):
    tokens, d = x.shape
    e = W.shape[0]
    b2 = b.reshape(1, e)
    logits_t = pl.pallas_call(
        _router_block,
        grid=(tokens // _BT,),
        in_specs=[
            pl.BlockSpec((_BT, d), lambda i: (i, 0)),
            pl.BlockSpec((e, d), lambda i: (0, 0)),
            pl.BlockSpec((1, e), lambda i: (0, 0)),
        ],
        out_specs=pl.BlockSpec((e, _BT), lambda i: (0, i)),
        out_shape=jax.ShapeDtypeStruct((e, tokens), jnp.float32),
    )(x, W, b2)
    return logits_t.T
